# Initial kernel scaffold; baseline (speedup 1.0000x reference)
import jax
import jax.numpy as jnp
from jax.experimental import pallas as pl


def kernel(x, edge_index, W1, b1, W2, b2):
    raise NotImplementedError("write your pallas kernel here")


# speedup vs baseline: 3.0812x; 3.0812x over previous
import functools

import jax
import jax.numpy as jnp
from jax.experimental import pallas as pl
from jax.experimental.pallas import tpu as pltpu

N = 10000
E = 320000
D_IN = 128
D_HID = 128
N_CLS = 8

_ROWS = 1000


def _k1_body(x_ref, w_ref, dinv_ref, g_ref):
    h = jnp.dot(x_ref[...], w_ref[...], preferred_element_type=jnp.float32)
    g_ref[...] = h * dinv_ref[...]


def _k2_body(s_ref, g_ref, dinv_ref, b_ref, w2_ref, g2_ref):
    z = jnp.maximum((s_ref[...] + g_ref[...]) * dinv_ref[...] + b_ref[...], 0.0)
    h2 = jnp.dot(z, w2_ref[...], preferred_element_type=jnp.float32)
    g2_ref[...] = h2 * dinv_ref[...]


def _k3_body(s_ref, g_ref, dinv_ref, b_ref, out_ref):
    out_ref[...] = (s_ref[...] + g_ref[...]) * dinv_ref[...] + b_ref[...]


def _rows_spec(d):
    return pl.BlockSpec((_ROWS, d), lambda i: (i, 0))


def _full_spec(shape):
    return pl.BlockSpec(shape, lambda i: (0,) * len(shape))


def _tc1(x, w, dinv):
    return pl.pallas_call(
        _k1_body,
        grid=(N // _ROWS,),
        in_specs=[_rows_spec(D_IN), _full_spec(w.shape), _rows_spec(1)],
        out_specs=_rows_spec(w.shape[1]),
        out_shape=jax.ShapeDtypeStruct((N, w.shape[1]), jnp.float32),
    )(x, w, dinv)


def _tc2(s, g, dinv, b, w2):
    d = g.shape[1]
    return pl.pallas_call(
        _k2_body,
        grid=(N // _ROWS,),
        in_specs=[_rows_spec(d), _rows_spec(d), _rows_spec(1),
                  _full_spec((1, d)), _full_spec(w2.shape)],
        out_specs=_rows_spec(w2.shape[1]),
        out_shape=jax.ShapeDtypeStruct((N, w2.shape[1]), jnp.float32),
    )(s, g, dinv, b.reshape(1, d), w2)


def _tc3(s, g, dinv, b):
    d = g.shape[1]
    return pl.pallas_call(
        _k3_body,
        grid=(N // _ROWS,),
        in_specs=[_rows_spec(d), _rows_spec(d), _rows_spec(1),
                  _full_spec((1, d))],
        out_specs=_rows_spec(d),
        out_shape=jax.ShapeDtypeStruct((N, d), jnp.float32),
    )(s, g, dinv, b.reshape(1, d))


def kernel(x, edge_index, W1, b1, W2, b2):
    src = edge_index[0]
    dst = edge_index[1]
    deg = jnp.zeros((N,), jnp.float32).at[dst].add(1.0) + 1.0
    dinv = jax.lax.rsqrt(deg).reshape(N, 1)

    g1 = _tc1(x, W1, dinv)
    s1 = jnp.zeros((N, D_HID), jnp.float32).at[dst].add(g1[src])
    g2 = _tc2(s1, g1, dinv, b1, W2)
    s2 = jnp.zeros((N, N_CLS), jnp.float32).at[dst].add(g2[src])
    return _tc3(s2, g2, dinv, b2)


# speedup vs baseline: 22.0846x; 7.1674x over previous
import functools

import jax
import jax.numpy as jnp
from jax import lax
from jax.experimental import pallas as pl
from jax.experimental.pallas import tpu as pltpu
from jax.experimental.pallas import tpu_sc as plsc

N = 10000
NPAD = 10240
E = 320000
D_IN = 128
D_HID = 128
N_CLS = 8
N_CLS_PAD = 16

NW = 32
EPT = E // NW
KB = 125
NBLK = EPT // KB
RPT = NPAD // 16

_ROWS = 1000

_mesh = plsc.VectorSubcoreMesh(core_axis_name="c", subcore_axis_name="s")
_sc_params = pltpu.CompilerParams(needs_layout_passes=False)


@functools.partial(
    pl.kernel,
    mesh=_mesh,
    out_type=jax.ShapeDtypeStruct((NW, NPAD), jnp.float32),
    compiler_params=_sc_params,
    scratch_types=[
        pltpu.VMEM((EPT,), jnp.int32),
        pltpu.VMEM((NPAD,), jnp.float32),
    ],
)
def _sc_deg(dst_hbm, out_hbm, dst_v, deg_v):
    c = lax.axis_index("c")
    s = lax.axis_index("s")
    w = c * 16 + s
    zero16 = jnp.zeros((16,), jnp.float32)

    def zbody(i, carry):
        deg_v[pl.ds(i * 16, 16)] = zero16
        return carry

    lax.fori_loop(0, NPAD // 16, zbody, 0)
    pltpu.sync_copy(dst_hbm.at[pl.ds(w * EPT, EPT)], dst_v)
    one16 = jnp.ones((16,), jnp.float32)

    def body(i, carry):
        idx = dst_v[pl.ds(i * 16, 16)]
        plsc.addupdate_scatter(deg_v, [idx], one16)
        return carry

    lax.fori_loop(0, EPT // 16, body, 0)
    pltpu.sync_copy(deg_v, out_hbm.at[w])


def _sc_scatter_body(src_hbm, dst_hbm, g_hbm, zeros_hbm, out_hbm,
                     src_v, dst_v, stage, acc):
    c = lax.axis_index("c")
    s = lax.axis_index("s")
    pltpu.sync_copy(src_hbm.at[c, s], src_v)
    pltpu.sync_copy(dst_hbm.at[c, s], dst_v)
    pltpu.sync_copy(zeros_hbm.at[pl.ds(s * RPT, RPT)],
                    acc.at[pl.ds(s * RPT, RPT)])
    plsc.subcore_barrier()

    def body(j, carry):
        pltpu.sync_copy(g_hbm.at[src_v.at[j]], stage)
        pltpu.sync_copy(stage, acc.at[dst_v.at[j]], add=True)
        return carry

    lax.fori_loop(0, NBLK, body, 0)
    plsc.subcore_barrier()
    pltpu.sync_copy(acc.at[pl.ds(s * RPT, RPT)],
                    out_hbm.at[c, pl.ds(s * RPT, RPT)])


def _make_sc_scatter(d):
    return functools.partial(
        pl.kernel,
        mesh=_mesh,
        out_type=jax.ShapeDtypeStruct((2, NPAD, d), jnp.float32),
        compiler_params=_sc_params,
        scratch_types=[
            pltpu.VMEM((NBLK, KB), jnp.int32),
            pltpu.VMEM((NBLK, KB), jnp.int32),
            pltpu.VMEM((KB, d), jnp.float32),
            pltpu.VMEM_SHARED((NPAD, d), jnp.float32),
        ],
    )(_sc_scatter_body)


_sc_scatter_128 = _make_sc_scatter(D_HID)


def _dinv_body(degp_ref, dinv_ref):
    deg = jnp.sum(degp_ref[...], axis=0) + 1.0
    dinv_ref[...] = lax.rsqrt(deg)[:, None]


def _k1_body(x_ref, w_ref, dinv_ref, g_ref):
    h = jnp.dot(x_ref[...], w_ref[...], preferred_element_type=jnp.float32)
    g_ref[...] = h * dinv_ref[...]


def _k2_body(s_ref, g_ref, dinv_ref, b_ref, gz_ref):
    agg = s_ref[0] + s_ref[1] + g_ref[...]
    z = jnp.maximum(agg * dinv_ref[...] + b_ref[...], 0.0)
    gz_ref[...] = z * dinv_ref[...]


def _k3_body(s_ref, g_ref, dinv_ref, b_ref, w2_ref, out_ref):
    agg = (s_ref[0] + s_ref[1] + g_ref[...]) * dinv_ref[...]
    out_ref[...] = (
        jnp.dot(agg, w2_ref[...], preferred_element_type=jnp.float32)
        + b_ref[...]
    )


def _rows_spec(d):
    return pl.BlockSpec((_ROWS, d), lambda i: (i, 0))


def _parts_spec(d):
    return pl.BlockSpec((2, _ROWS, d), lambda i: (0, i, 0))


def _full_spec(shape):
    return pl.BlockSpec(shape, lambda i: (0,) * len(shape))


def _dinv_tc(deg_parts):
    return pl.pallas_call(
        _dinv_body,
        grid=(NPAD // 1280,),
        in_specs=[pl.BlockSpec((NW, 1280), lambda i: (0, i))],
        out_specs=pl.BlockSpec((1280, 1), lambda i: (i, 0)),
        out_shape=jax.ShapeDtypeStruct((NPAD, 1), jnp.float32),
    )(deg_parts)


def _tc1(x, w, dinv):
    return pl.pallas_call(
        _k1_body,
        grid=(N // _ROWS,),
        in_specs=[_rows_spec(D_IN), _full_spec(w.shape), _rows_spec(1)],
        out_specs=_rows_spec(w.shape[1]),
        out_shape=jax.ShapeDtypeStruct((N, w.shape[1]), jnp.float32),
    )(x, w, dinv)


def _tc2(s_parts, g, dinv, b):
    d = g.shape[1]
    return pl.pallas_call(
        _k2_body,
        grid=(N // _ROWS,),
        in_specs=[_parts_spec(d), _rows_spec(d), _rows_spec(1),
                  _full_spec((1, d))],
        out_specs=_rows_spec(d),
        out_shape=jax.ShapeDtypeStruct((N, d), jnp.float32),
    )(s_parts, g, dinv, b.reshape(1, d))


def _tc3(s_parts, g, dinv, b, w2):
    d = g.shape[1]
    return pl.pallas_call(
        _k3_body,
        grid=(N // _ROWS,),
        in_specs=[_parts_spec(d), _rows_spec(d), _rows_spec(1),
                  _full_spec((1, N_CLS)), _full_spec(w2.shape)],
        out_specs=pl.BlockSpec((_ROWS, N_CLS), lambda i: (i, 0)),
        out_shape=jax.ShapeDtypeStruct((N, N_CLS), jnp.float32),
    )(s_parts, g, dinv, b.reshape(1, N_CLS), w2)


def kernel(x, edge_index, W1, b1, W2, b2):
    src = edge_index[0]
    dst = edge_index[1]
    srcr = src.reshape(2, 16, NBLK, KB)
    dstr = dst.reshape(2, 16, NBLK, KB)

    deg_parts = _sc_deg(dst)
    dinv = _dinv_tc(deg_parts)[:N]

    g1 = _tc1(x, W1, dinv)
    s1p = _sc_scatter_128(srcr, dstr, g1, jnp.zeros((NPAD, D_HID), jnp.float32))

    gz = _tc2(s1p, g1, dinv, b1)
    s2p = _sc_scatter_128(srcr, dstr, gz, jnp.zeros((NPAD, D_HID), jnp.float32))
    return _tc3(s2p, gz, dinv, b2, W2)
